# speedup vs baseline: 2.0680x; 2.0680x over previous
import functools

import jax
import jax.numpy as jnp
from jax import lax
from jax.experimental import pallas as pl
from jax.experimental.pallas import tpu as pltpu
from jax.experimental.pallas import tpu_sc as plsc

_D = 128
_CHUNK = 128
_NW = 32
_STEPS = 200
_BPW = _STEPS * _CHUNK


def _make_sc_gather():
    mesh = plsc.VectorSubcoreMesh(core_axis_name="c", subcore_axis_name="s")

    @functools.partial(
        pl.kernel,
        mesh=mesh,
        out_type=jax.ShapeDtypeStruct((_NW, _BPW, _D), jnp.float32),
        scratch_types=[
            pltpu.VMEM((_STEPS, _CHUNK), jnp.int32),
            pltpu.VMEM((_CHUNK, _D), jnp.float32),
            pltpu.SemaphoreType.DMA,
        ],
    )
    def sc_gather(table_hbm, idx_hbm, out_hbm, idx_v, rows_v, sem):
        wid = lax.axis_index("s") * 2 + lax.axis_index("c")
        pltpu.sync_copy(idx_hbm.at[wid], idx_v)

        def step(j, carry):
            pltpu.async_copy(table_hbm.at[idx_v.at[j]], rows_v, sem).wait()
            pltpu.sync_copy(rows_v, out_hbm.at[wid, pl.ds(j * _CHUNK, _CHUNK)])
            return carry

        lax.fori_loop(0, _STEPS, step, None)

    return sc_gather


_sc_gather = _make_sc_gather()


def kernel(indices, embeddings):
    b, l = indices.shape
    idx = indices.reshape(_NW, _STEPS, _CHUNK)
    out = _sc_gather(embeddings, idx)
    return out.reshape(b, l, _D)


# speedup vs baseline: 2.0815x; 1.0066x over previous
import functools

import jax
import jax.numpy as jnp
from jax import lax
from jax.experimental import pallas as pl
from jax.experimental.pallas import tpu as pltpu
from jax.experimental.pallas import tpu_sc as plsc

_D = 128
_CHUNK = 128
_K = 2
_GROUP = _K * _CHUNK
_NW = 32
_STEPS = 200
_PHASES = _STEPS // _K
_Q = _PHASES // 2
_BPW = _STEPS * _CHUNK


def _make_sc_gather():
    mesh = plsc.VectorSubcoreMesh(core_axis_name="c", subcore_axis_name="s")

    @functools.partial(
        pl.kernel,
        mesh=mesh,
        out_type=jax.ShapeDtypeStruct((_NW, _BPW, _D), jnp.float32),
        scratch_types=[
            pltpu.VMEM((_STEPS, _CHUNK), jnp.int32),
            pltpu.VMEM((2, _GROUP, _D), jnp.float32),
            pltpu.SemaphoreType.DMA,
            pltpu.SemaphoreType.DMA,
            pltpu.SemaphoreType.DMA,
        ],
    )
    def sc_gather(table_hbm, idx_hbm, out_hbm, idx_v, rows_v, g_sem, w_sem0,
                  w_sem1):
        wid = lax.axis_index("s") * 2 + lax.axis_index("c")
        pltpu.sync_copy(idx_hbm.at[wid], idx_v)

        def issue_gathers(phase, grp):
            for b in range(_K):
                pltpu.async_copy(
                    table_hbm.at[idx_v.at[phase * _K + b]],
                    rows_v.at[grp, pl.ds(b * _CHUNK, _CHUNK)],
                    g_sem,
                )

        def drain_gathers(grp):
            for b in range(_K):
                pltpu.make_async_copy(
                    table_hbm.at[idx_v.at[0]],
                    rows_v.at[grp, pl.ds(b * _CHUNK, _CHUNK)],
                    g_sem,
                ).wait()

        def write_group(phase, grp, w_sem):
            pltpu.async_copy(
                rows_v.at[grp],
                out_hbm.at[wid, pl.ds(phase * _GROUP, _GROUP)],
                w_sem,
            )

        def drain_write(grp, w_sem):
            pltpu.make_async_copy(
                rows_v.at[grp],
                out_hbm.at[wid, pl.ds(0, _GROUP)],
                w_sem,
            ).wait()

        issue_gathers(0, 0)

        def qstep(q, carry):
            p0 = 2 * q
            drain_gathers(0)
            write_group(p0, 0, w_sem0)

            @pl.when(q > 0)
            def _():
                drain_write(1, w_sem1)

            issue_gathers(p0 + 1, 1)

            drain_gathers(1)
            write_group(p0 + 1, 1, w_sem1)
            drain_write(0, w_sem0)

            @pl.when(q < _Q - 1)
            def _():
                issue_gathers(p0 + 2, 0)

            return carry

        lax.fori_loop(0, _Q, qstep, None)
        drain_write(1, w_sem1)

    return sc_gather


_sc_gather = _make_sc_gather()


def kernel(indices, embeddings):
    b, l = indices.shape
    idx = indices.reshape(_NW, _STEPS, _CHUNK)
    out = _sc_gather(embeddings, idx)
    return out.reshape(b, l, _D)
